# baseline (speedup 1.0000x reference)
import functools

import jax
import jax.numpy as jnp
from jax import lax
from jax.experimental import pallas as pl
from jax.experimental.pallas import tpu as pltpu
from jax.experimental.pallas import tpu_sc as plsc

ATTRS_ = (25, 6, 18, 3, 9, 6, 4, 5, 5, 3, 3, 3, 3, 3, 10)
DIMS_ = (10, 3, 9, 3, 5, 3, 2, 3, 3, 2, 2, 2, 2, 2, 5)
B_ = 16384
OUT_W = 1 + sum(DIMS_)

_XCOL = []
_FBASE = []
_fb = 0
for _i in range(15):
    for _d in range(DIMS_[_i]):
        _XCOL.append(_i + 1)
        _FBASE.append(_fb)
        _fb += ATTRS_[_i]
FLAT_LEN = _fb
FLAT_PAD = ((FLAT_LEN + 7) // 8) * 8

NW = 32
ROWS_PER_W = B_ // NW
L = 16
NBLK = ROWS_PER_W // L


def _body(xt_hbm, flat_hbm, out_hbm, xt_v, flat_v, out_v):
    wid = lax.axis_index("s") * 2 + lax.axis_index("c")
    base = wid * ROWS_PER_W
    pltpu.sync_copy(flat_hbm, flat_v)
    pltpu.sync_copy(xt_hbm.at[:, pl.ds(base, ROWS_PER_W)], xt_v)

    def block(b, _):
        rr = b * L
        sl = pl.ds(rr, L)
        out_v[0, sl] = xt_v[0, sl]
        idxv = [None] + [xt_v[j, sl].astype(jnp.int32) for j in range(1, 16)]
        for c in range(OUT_W - 1):
            v = plsc.load_gather(flat_v, [idxv[_XCOL[c]] + _FBASE[c]])
            out_v[c + 1, sl] = v
        return _

    lax.fori_loop(0, NBLK, block, None)
    pltpu.sync_copy(out_v, out_hbm.at[:, pl.ds(base, ROWS_PER_W)])


@functools.partial(jax.jit, static_argnames=("interpret",))
def kernel(x, tables, interpret=False):
    flat = jnp.concatenate(
        [t.T.ravel() for t in tables]
        + [jnp.zeros((FLAT_PAD - FLAT_LEN,), jnp.float32)]
    )
    run = pl.kernel(
        _body,
        out_type=jax.ShapeDtypeStruct((OUT_W, B_), jnp.float32),
        mesh=plsc.VectorSubcoreMesh(
            core_axis_name="c", subcore_axis_name="s",
            num_cores=2, num_subcores=16,
        ),
        scratch_types=[
            pltpu.VMEM((16, ROWS_PER_W), jnp.float32),
            pltpu.VMEM((FLAT_PAD,), jnp.float32),
            pltpu.VMEM((OUT_W, ROWS_PER_W), jnp.float32),
        ],
        compiler_params=pltpu.CompilerParams(
            needs_layout_passes=False, use_tc_tiling_on_sc=True
        ),
        interpret=interpret,
    )
    return run(x.T, flat).T


# speedup vs baseline: 1.0182x; 1.0182x over previous
import functools

import jax
import jax.numpy as jnp
from jax import lax
from jax.experimental import pallas as pl
from jax.experimental.pallas import tpu as pltpu
from jax.experimental.pallas import tpu_sc as plsc

ATTRS_ = (25, 6, 18, 3, 9, 6, 4, 5, 5, 3, 3, 3, 3, 3, 10)
DIMS_ = (10, 3, 9, 3, 5, 3, 2, 3, 3, 2, 2, 2, 2, 2, 5)
B_ = 16384
OUT_W = 1 + sum(DIMS_)
TW = 16
TROWS = sum(ATTRS_)

_XCOL = []
_DCOL = []
_RBASE = []
_rb = 0
for _i in range(15):
    for _d in range(DIMS_[_i]):
        _XCOL.append(_i + 1)
        _DCOL.append(_d)
        _RBASE.append(_rb)
    _rb += ATTRS_[_i]

NW = 32
ROWS_PER_W = B_ // NW
L = 16
NBLK = ROWS_PER_W // L


def _body(xt_hbm, tbl_hbm, out_hbm, xt_v, tbl_v, out_v):
    wid = lax.axis_index("s") * 2 + lax.axis_index("c")
    base = wid * ROWS_PER_W
    pltpu.sync_copy(tbl_hbm, tbl_v)
    pltpu.sync_copy(xt_hbm.at[:, pl.ds(base, ROWS_PER_W)], xt_v)

    dsplat = [jnp.full((L,), d, jnp.int32) for d in range(max(DIMS_))]

    def block(b, _):
        rr = b * L
        sl = pl.ds(rr, L)
        out_v[0, sl] = xt_v[0, sl]
        ridx = [None] * 16
        for j in range(1, 16):
            ridx[j] = xt_v[j, sl].astype(jnp.int32)
        for c in range(OUT_W - 1):
            v = plsc.load_gather(
                tbl_v, [dsplat[_DCOL[c]], ridx[_XCOL[c]] + _RBASE[c]]
            )
            out_v[c + 1, sl] = v
        return _

    lax.fori_loop(0, NBLK, block, None)
    pltpu.sync_copy(out_v, out_hbm.at[:, pl.ds(base, ROWS_PER_W)])


@functools.partial(jax.jit, static_argnames=("interpret",))
def kernel(x, tables, interpret=False):
    stacked = jnp.concatenate(
        [jnp.pad(t, ((0, 0), (0, TW - t.shape[1]))) for t in tables]
    )
    run = pl.kernel(
        _body,
        out_type=jax.ShapeDtypeStruct((OUT_W, B_), jnp.float32),
        mesh=plsc.VectorSubcoreMesh(
            core_axis_name="c", subcore_axis_name="s",
            num_cores=2, num_subcores=16,
        ),
        scratch_types=[
            pltpu.VMEM((16, ROWS_PER_W), jnp.float32),
            pltpu.VMEM((TW, TROWS), jnp.float32),
            pltpu.VMEM((OUT_W, ROWS_PER_W), jnp.float32),
        ],
        compiler_params=pltpu.CompilerParams(
            needs_layout_passes=False, use_tc_tiling_on_sc=True
        ),
        interpret=interpret,
    )
    return run(x.T, stacked.T).T


# speedup vs baseline: 1.2500x; 1.2277x over previous
import functools

import jax
import jax.numpy as jnp
from jax import lax
from jax.experimental import pallas as pl
from jax.experimental.pallas import tpu as pltpu
from jax.experimental.pallas import tpu_sc as plsc

ATTRS_ = (25, 6, 18, 3, 9, 6, 4, 5, 5, 3, 3, 3, 3, 3, 10)
DIMS_ = (10, 3, 9, 3, 5, 3, 2, 3, 3, 2, 2, 2, 2, 2, 5)
B_ = 16384
OUT_W = 1 + sum(DIMS_)

NW = 32
ROWS_PER_W = B_ // NW
L = 16
NBLK = ROWS_PER_W // L


def _body(xt_hbm, *refs):
    tt_hbm = refs[:15]
    out_hbm = refs[15]
    xt_v = refs[16]
    tt_v = refs[17:32]
    out_v = refs[32]
    sem = refs[33]

    wid = lax.axis_index("s") * 2 + lax.axis_index("c")
    base = wid * ROWS_PER_W
    copies = [pltpu.async_copy(tt_hbm[i], tt_v[i], sem) for i in range(15)]
    copies.append(
        pltpu.async_copy(xt_hbm.at[:, pl.ds(base, ROWS_PER_W)], xt_v, sem)
    )
    for c in copies:
        c.wait()

    dsplat = [jnp.full((L,), d, jnp.int32) for d in range(max(DIMS_))]

    def block(b, _):
        rr = b * L
        sl = pl.ds(rr, L)
        out_v[0, sl] = xt_v[0, sl]
        ridx = [None] * 16
        for j in range(1, 16):
            ridx[j] = xt_v[j, sl].astype(jnp.int32)
        c = 1
        for i in range(15):
            for d in range(DIMS_[i]):
                out_v[c, sl] = plsc.load_gather(
                    tt_v[i], [dsplat[d], ridx[i + 1]]
                )
                c += 1
        return _

    lax.fori_loop(0, NBLK, block, None)
    pltpu.sync_copy(out_v, out_hbm.at[:, pl.ds(base, ROWS_PER_W)])


@functools.partial(jax.jit, static_argnames=("interpret",))
def kernel(x, tables, interpret=False):
    run = pl.kernel(
        _body,
        out_type=jax.ShapeDtypeStruct((OUT_W, B_), jnp.float32),
        mesh=plsc.VectorSubcoreMesh(
            core_axis_name="c", subcore_axis_name="s",
            num_cores=2, num_subcores=16,
        ),
        scratch_types=[
            pltpu.VMEM((16, ROWS_PER_W), jnp.float32),
            *[
                pltpu.VMEM((DIMS_[i], ATTRS_[i]), jnp.float32)
                for i in range(15)
            ],
            pltpu.VMEM((OUT_W, ROWS_PER_W), jnp.float32),
            pltpu.SemaphoreType.DMA,
        ],
        compiler_params=pltpu.CompilerParams(
            needs_layout_passes=False, use_tc_tiling_on_sc=True
        ),
        interpret=interpret,
    )
    return run(x.T, *[t.T for t in tables]).T
